# Initial kernel scaffold; baseline (speedup 1.0000x reference)
import functools

import jax
import jax.numpy as jnp
from jax import lax
from jax.experimental import pallas as pl
from jax.experimental.pallas import tpu as pltpu
from jax.experimental.pallas import tpu_sc as plsc

N = 10000
E = 320000
D = 64
NC = 2
NS = 16
NW = NC * NS
EPT = E // NW
CH = 100
NCH = EPT // CH
RPT = N // NS
ZR = 125

_P = lax.Precision.HIGHEST


def _lk(v):
    return jnp.where(v >= 0, v, 0.01 * v)


def _stage1_body(f_ref, id_ref, mw_ref, mb_ref, c1_ref, l1_ref, lb_ref,
                 x_ref, xw_ref, xh_ref):
    t = jnp.dot(f_ref[...], mw_ref[...], precision=_P) + mb_ref[...]
    nrm = jnp.sqrt(jnp.sum(t * t, axis=1, keepdims=True))
    x = t / jnp.maximum(nrm, 1e-12)
    x_ref[...] = x
    xw_ref[...] = jnp.dot(x, c1_ref[...], precision=_P)
    xh_ref[...] = _lk(jnp.dot(x, l1_ref[...], precision=_P) + lb_ref[...]) + id_ref[...]


def _stage1(feat, ide, mlp_wT, mlp_b, conv1_w, lin1_wT, lin1_b):
    out = [jax.ShapeDtypeStruct((N, D), jnp.float32)] * 3
    return pl.pallas_call(_stage1_body, out_shape=out)(
        feat, ide, mlp_wT, mlp_b, conv1_w, lin1_wT, lin1_b)


def _stage2_body(p_ref, xh_ref, id_ref, gw_ref, gb_ref, cw_ref, lw_ref, lb_ref,
                 xw_ref, xh2_ref):
    h = _lk(p_ref[0] + p_ref[1])
    x2 = _lk(jnp.dot(h, gw_ref[...], precision=_P) + gb_ref[...] + xh_ref[...])
    xw_ref[...] = jnp.dot(x2, cw_ref[...], precision=_P)
    xh2_ref[...] = _lk(jnp.dot(x2, lw_ref[...], precision=_P) + lb_ref[...]) + id_ref[...]


def _stage2(p, xh, ide, g_wT, g_b, conv_w, lin_wT, lin_b):
    out = [jax.ShapeDtypeStruct((N, D), jnp.float32)] * 2
    return pl.pallas_call(_stage2_body, out_shape=out)(
        p, xh, ide, g_wT, g_b, conv_w, lin_wT, lin_b)


def _stage3_body(p_ref, xh_ref, gw_ref, gb_ref, o_ref):
    h = _lk(p_ref[0] + p_ref[1])
    o_ref[...] = _lk(jnp.dot(h, gw_ref[...], precision=_P) + gb_ref[...] + xh_ref[...])


def _stage3(p, xh2, g_wT, g_b):
    out = jax.ShapeDtypeStruct((N, D), jnp.float32)
    return pl.pallas_call(_stage3_body, out_shape=out)(p, xh2, g_wT, g_b)


def _sc_conv(xw, src3, dst3):
    mesh = plsc.VectorSubcoreMesh(core_axis_name="c", subcore_axis_name="s")

    @functools.partial(
        pl.kernel,
        out_type=jax.ShapeDtypeStruct((NC, N, D), jnp.float32),
        mesh=mesh,
        scratch_types=[
            pltpu.VMEM((NCH, CH), jnp.int32),
            pltpu.VMEM((NCH, CH), jnp.int32),
            pltpu.VMEM((CH, D), jnp.float32),
            pltpu.VMEM((ZR, D), jnp.float32),
            pltpu.VMEM_SHARED((N, D), jnp.float32),
            pltpu.SemaphoreType.DMA,
        ],
    )
    def k(xw_hbm, src_hbm, dst_hbm, out_hbm, src_v, dst_v, rows_v, zb_v, acc_sh, sem):
        core = lax.axis_index("c")
        sub = lax.axis_index("s")
        wid = core * NS + sub

        pltpu.async_copy(src_hbm.at[wid], src_v, sem).wait()
        pltpu.async_copy(dst_hbm.at[wid], dst_v, sem).wait()

        @pl.loop(0, ZR)
        def _(i):
            for j in range(D // 16):
                zb_v[i, pl.ds(j * 16, 16)] = jnp.zeros((16,), jnp.float32)

        for r in range(RPT // ZR):
            pltpu.sync_copy(zb_v, acc_sh.at[pl.ds(sub * RPT + r * ZR, ZR)])
        plsc.subcore_barrier()

        @pl.loop(0, NCH)
        def _(ci):
            pltpu.async_copy(xw_hbm.at[src_v.at[ci]], rows_v, sem).wait()
            pltpu.sync_copy(rows_v, acc_sh.at[dst_v.at[ci]], add=True)

        plsc.subcore_barrier()
        for r in range(RPT // ZR):
            sl = pl.ds(sub * RPT + r * ZR, ZR)
            pltpu.sync_copy(acc_sh.at[sl], out_hbm.at[core].at[sl])

    return k(xw, src3, dst3)


@jax.jit
def _pipeline(features, id_embedding, edge_index, mlp_w, mlp_b, conv1_w,
              lin1_w, lin1_b, g1_w, g1_b, conv2_w, lin2_w, lin2_b, g2_w, g2_b):
    src3 = edge_index[0].reshape(NW, NCH, CH)
    dst3 = edge_index[1].reshape(NW, NCH, CH)
    mb = mlp_b.reshape(1, D)
    l1b = lin1_b.reshape(1, D)
    g1b = g1_b.reshape(1, D)
    l2b = lin2_b.reshape(1, D)
    g2b = g2_b.reshape(1, D)

    x, xw1, xh1 = _stage1(features, id_embedding, mlp_w.T, mb, conv1_w,
                          lin1_w.T, l1b)
    p1 = _sc_conv(xw1, src3, dst3)
    xw2, xh2 = _stage2(p1, xh1, id_embedding, g1_w.T, g1b, conv2_w,
                       lin2_w.T, l2b)
    p2 = _sc_conv(xw2, src3, dst3)
    return _stage3(p2, xh2, g2_w.T, g2b)


def kernel(features, id_embedding, edge_index, mlp_w, mlp_b, conv1_w,
           lin1_w, lin1_b, g1_w, g1_b, conv2_w, lin2_w, lin2_b, g2_w, g2_b):
    return _pipeline(features, id_embedding, edge_index, mlp_w, mlp_b, conv1_w,
                     lin1_w, lin1_b, g1_w, g1_b, conv2_w, lin2_w, lin2_b,
                     g2_w, g2_b)


# speedup vs baseline: 7.4137x; 7.4137x over previous
import functools

import jax
import jax.numpy as jnp
from jax import lax
from jax.experimental import pallas as pl
from jax.experimental.pallas import tpu as pltpu
from jax.experimental.pallas import tpu_sc as plsc

N = 10000
E = 320000
D = 64
NC = 2
NS = 16
NW = NC * NS
EPT = E // NW
CH = 100
NCH = EPT // CH
NP = 10240
RPT = NP // NS
ZR = 128

_P = lax.Precision.HIGHEST


def _lk(v):
    return jnp.where(v >= 0, v, 0.01 * v)


def _stage1_body(f_ref, id_ref, mw_ref, mb_ref, c1_ref, l1_ref, lb_ref,
                 x_ref, xw_ref, xh_ref):
    t = jnp.dot(f_ref[...], mw_ref[...], precision=_P) + mb_ref[...]
    nrm = jnp.sqrt(jnp.sum(t * t, axis=1, keepdims=True))
    x = t / jnp.maximum(nrm, 1e-12)
    x_ref[...] = x
    xw_ref[...] = jnp.dot(x, c1_ref[...], precision=_P)
    xh_ref[...] = _lk(jnp.dot(x, l1_ref[...], precision=_P) + lb_ref[...]) + id_ref[...]


def _stage1(feat, ide, mlp_wT, mlp_b, conv1_w, lin1_wT, lin1_b):
    out = [jax.ShapeDtypeStruct((N, D), jnp.float32)] * 3
    return pl.pallas_call(_stage1_body, out_shape=out)(
        feat, ide, mlp_wT, mlp_b, conv1_w, lin1_wT, lin1_b)


def _stage2_body(p_ref, xh_ref, id_ref, gw_ref, gb_ref, cw_ref, lw_ref, lb_ref,
                 xw_ref, xh2_ref):
    h = _lk(p_ref[0] + p_ref[1])[:N]
    x2 = _lk(jnp.dot(h, gw_ref[...], precision=_P) + gb_ref[...] + xh_ref[...])
    xw_ref[...] = jnp.dot(x2, cw_ref[...], precision=_P)
    xh2_ref[...] = _lk(jnp.dot(x2, lw_ref[...], precision=_P) + lb_ref[...]) + id_ref[...]


def _stage2(p, xh, ide, g_wT, g_b, conv_w, lin_wT, lin_b):
    out = [jax.ShapeDtypeStruct((N, D), jnp.float32)] * 2
    return pl.pallas_call(_stage2_body, out_shape=out)(
        p, xh, ide, g_wT, g_b, conv_w, lin_wT, lin_b)


def _stage3_body(p_ref, xh_ref, gw_ref, gb_ref, o_ref):
    h = _lk(p_ref[0] + p_ref[1])[:N]
    o_ref[...] = _lk(jnp.dot(h, gw_ref[...], precision=_P) + gb_ref[...] + xh_ref[...])


def _stage3(p, xh2, g_wT, g_b):
    out = jax.ShapeDtypeStruct((N, D), jnp.float32)
    return pl.pallas_call(_stage3_body, out_shape=out)(p, xh2, g_wT, g_b)


def _sc_conv(xw, src3, dst3):
    mesh = plsc.VectorSubcoreMesh(core_axis_name="c", subcore_axis_name="s")

    @functools.partial(
        pl.kernel,
        out_type=jax.ShapeDtypeStruct((NC, NP, D), jnp.float32),
        mesh=mesh,
        compiler_params=pltpu.CompilerParams(use_tc_tiling_on_sc=False),
        scratch_types=[
            pltpu.VMEM((NCH, CH), jnp.int32),
            pltpu.VMEM((NCH, CH), jnp.int32),
            pltpu.VMEM((CH, D), jnp.float32),
            pltpu.VMEM((ZR, D), jnp.float32),
            pltpu.VMEM_SHARED((NP, D), jnp.float32),
            pltpu.SemaphoreType.DMA,
        ],
    )
    def k(xw_hbm, src_hbm, dst_hbm, out_hbm, src_v, dst_v, rows_v, zb_v, acc_sh, sem):
        core = lax.axis_index("c")
        sub = lax.axis_index("s")
        wid = core * NS + sub

        pltpu.async_copy(src_hbm.at[wid], src_v, sem).wait()
        pltpu.async_copy(dst_hbm.at[wid], dst_v, sem).wait()

        @pl.loop(0, ZR)
        def _(i):
            for j in range(D // 16):
                zb_v[i, pl.ds(j * 16, 16)] = jnp.zeros((16,), jnp.float32)

        for r in range(RPT // ZR):
            pltpu.sync_copy(zb_v, acc_sh.at[pl.ds(sub * RPT + r * ZR, ZR)])
        plsc.subcore_barrier()

        @pl.loop(0, NCH)
        def _(ci):
            pltpu.async_copy(xw_hbm.at[src_v.at[ci]], rows_v, sem).wait()
            pltpu.sync_copy(rows_v, acc_sh.at[dst_v.at[ci]], add=True)

        plsc.subcore_barrier()
        for r in range(RPT // ZR):
            sl = pl.ds(sub * RPT + r * ZR, ZR)
            pltpu.sync_copy(acc_sh.at[sl], out_hbm.at[core].at[sl])

    return k(xw, src3, dst3)


@jax.jit
def _pipeline(features, id_embedding, edge_index, mlp_w, mlp_b, conv1_w,
              lin1_w, lin1_b, g1_w, g1_b, conv2_w, lin2_w, lin2_b, g2_w, g2_b):
    src3 = edge_index[0].reshape(NW, NCH, CH)
    dst3 = edge_index[1].reshape(NW, NCH, CH)
    mb = mlp_b.reshape(1, D)
    l1b = lin1_b.reshape(1, D)
    g1b = g1_b.reshape(1, D)
    l2b = lin2_b.reshape(1, D)
    g2b = g2_b.reshape(1, D)

    x, xw1, xh1 = _stage1(features, id_embedding, mlp_w.T, mb, conv1_w,
                          lin1_w.T, l1b)
    p1 = _sc_conv(xw1, src3, dst3)
    xw2, xh2 = _stage2(p1, xh1, id_embedding, g1_w.T, g1b, conv2_w,
                       lin2_w.T, l2b)
    p2 = _sc_conv(xw2, src3, dst3)
    return _stage3(p2, xh2, g2_w.T, g2b)


def kernel(features, id_embedding, edge_index, mlp_w, mlp_b, conv1_w,
           lin1_w, lin1_b, g1_w, g1_b, conv2_w, lin2_w, lin2_b, g2_w, g2_b):
    return _pipeline(features, id_embedding, edge_index, mlp_w, mlp_b, conv1_w,
                     lin1_w, lin1_b, g1_w, g1_b, conv2_w, lin2_w, lin2_b,
                     g2_w, g2_b)


# speedup vs baseline: 11.6801x; 1.5755x over previous
import functools

import jax
import jax.numpy as jnp
from jax import lax
from jax.experimental import pallas as pl
from jax.experimental.pallas import tpu as pltpu
from jax.experimental.pallas import tpu_sc as plsc

N = 10000
E = 320000
D = 64
NC = 2
NS = 16
NW = NC * NS
EPT = E // NW
CH = 100
NCH = EPT // CH
NP = 10240
RPT = NP // NS
ZR = 128

_P = lax.Precision.HIGHEST


def _lk(v):
    return jnp.where(v >= 0, v, 0.01 * v)


def _stage1_body(f_ref, id_ref, mw_ref, mb_ref, c1_ref, l1_ref, lb_ref,
                 x_ref, xw_ref, xh_ref):
    t = jnp.dot(f_ref[...], mw_ref[...], precision=_P) + mb_ref[...]
    nrm = jnp.sqrt(jnp.sum(t * t, axis=1, keepdims=True))
    x = t / jnp.maximum(nrm, 1e-12)
    x_ref[...] = x
    xw_ref[...] = jnp.dot(x, c1_ref[...], precision=_P)
    xh_ref[...] = _lk(jnp.dot(x, l1_ref[...], precision=_P) + lb_ref[...]) + id_ref[...]


def _stage1(feat, ide, mlp_wT, mlp_b, conv1_w, lin1_wT, lin1_b):
    out = [jax.ShapeDtypeStruct((N, D), jnp.float32)] * 3
    return pl.pallas_call(_stage1_body, out_shape=out)(
        feat, ide, mlp_wT, mlp_b, conv1_w, lin1_wT, lin1_b)


def _stage2_body(p_ref, xh_ref, id_ref, gw_ref, gb_ref, cw_ref, lw_ref, lb_ref,
                 xw_ref, xh2_ref):
    h = _lk(p_ref[0] + p_ref[1])[:N]
    x2 = _lk(jnp.dot(h, gw_ref[...], precision=_P) + gb_ref[...] + xh_ref[...])
    xw_ref[...] = jnp.dot(x2, cw_ref[...], precision=_P)
    xh2_ref[...] = _lk(jnp.dot(x2, lw_ref[...], precision=_P) + lb_ref[...]) + id_ref[...]


def _stage2(p, xh, ide, g_wT, g_b, conv_w, lin_wT, lin_b):
    out = [jax.ShapeDtypeStruct((N, D), jnp.float32)] * 2
    return pl.pallas_call(_stage2_body, out_shape=out)(
        p, xh, ide, g_wT, g_b, conv_w, lin_wT, lin_b)


def _stage3_body(p_ref, xh_ref, gw_ref, gb_ref, o_ref):
    h = _lk(p_ref[0] + p_ref[1])[:N]
    o_ref[...] = _lk(jnp.dot(h, gw_ref[...], precision=_P) + gb_ref[...] + xh_ref[...])


def _stage3(p, xh2, g_wT, g_b):
    out = jax.ShapeDtypeStruct((N, D), jnp.float32)
    return pl.pallas_call(_stage3_body, out_shape=out)(p, xh2, g_wT, g_b)


def _sc_conv(xw, src3, dst3):
    mesh = plsc.VectorSubcoreMesh(core_axis_name="c", subcore_axis_name="s")

    @functools.partial(
        pl.kernel,
        out_type=jax.ShapeDtypeStruct((NC, NP, D), jnp.float32),
        mesh=mesh,
        compiler_params=pltpu.CompilerParams(use_tc_tiling_on_sc=False),
        scratch_types=[
            pltpu.VMEM((NCH, CH), jnp.int32),
            pltpu.VMEM((NCH, CH), jnp.int32),
            pltpu.VMEM((CH, D), jnp.float32),
            pltpu.VMEM((CH, D), jnp.float32),
            pltpu.VMEM((CH, D), jnp.float32),
            pltpu.VMEM((CH, D), jnp.float32),
            pltpu.VMEM((ZR, D), jnp.float32),
            pltpu.VMEM_SHARED((NP, D), jnp.float32),
            pltpu.SemaphoreType.DMA,
            pltpu.SemaphoreType.DMA,
            pltpu.SemaphoreType.DMA,
            pltpu.SemaphoreType.DMA,
            pltpu.SemaphoreType.DMA,
            pltpu.SemaphoreType.DMA,
            pltpu.SemaphoreType.DMA,
            pltpu.SemaphoreType.DMA,
            pltpu.SemaphoreType.DMA,
        ],
    )
    def k(xw_hbm, src_hbm, dst_hbm, out_hbm, src_v, dst_v, rb0, rb1, rb2, rb3,
          zb_v, acc_sh, g0, g1, g2, g3, s0, s1, s2, s3, sem):
        core = lax.axis_index("c")
        sub = lax.axis_index("s")
        wid = core * NS + sub
        rbufs = [rb0, rb1, rb2, rb3]
        gsem = [g0, g1, g2, g3]
        ssem = [s0, s1, s2, s3]

        pltpu.async_copy(src_hbm.at[wid], src_v, sem).wait()
        pltpu.async_copy(dst_hbm.at[wid], dst_v, sem).wait()

        @pl.loop(0, ZR)
        def _(i):
            for j in range(D // 16):
                zb_v[i, pl.ds(j * 16, 16)] = jnp.zeros((16,), jnp.float32)

        for r in range(RPT // ZR):
            pltpu.sync_copy(zb_v, acc_sh.at[pl.ds(sub * RPT + r * ZR, ZR)])
        plsc.subcore_barrier()

        def fire_g(c, b):
            pltpu.async_copy(xw_hbm.at[src_v.at[c]], rbufs[b], gsem[b])

        def wait_g(c, b):
            pltpu.make_async_copy(xw_hbm.at[src_v.at[c]], rbufs[b], gsem[b]).wait()

        def fire_s(c, b):
            pltpu.async_copy(rbufs[b], acc_sh.at[dst_v.at[c]], ssem[b], add=True)

        def wait_s(c, b):
            pltpu.make_async_copy(rbufs[b], acc_sh.at[dst_v.at[c]], ssem[b]).wait()

        fire_g(0, 0)
        fire_g(1, 1)

        @pl.loop(0, NCH, step=4)
        def _(ci):
            for j in range(4):
                c = ci + j
                bj = j
                bo = (j + 2) % 4
                if j < 2:
                    @pl.when(c >= 2)
                    def _():
                        wait_s(c - 2, bo)
                    fire_g(c + 2, bo)
                else:
                    wait_s(c - 2, bo)

                    @pl.when(c + 2 < NCH)
                    def _():
                        fire_g(c + 2, bo)
                wait_g(c, bj)
                fire_s(c, bj)

        wait_s(NCH - 2, (NCH - 2) % 4)
        wait_s(NCH - 1, (NCH - 1) % 4)
        plsc.subcore_barrier()
        for r in range(RPT // ZR):
            sl = pl.ds(sub * RPT + r * ZR, ZR)
            pltpu.sync_copy(acc_sh.at[sl], out_hbm.at[core].at[sl])

    return k(xw, src3, dst3)


@jax.jit
def _pipeline(features, id_embedding, edge_index, mlp_w, mlp_b, conv1_w,
              lin1_w, lin1_b, g1_w, g1_b, conv2_w, lin2_w, lin2_b, g2_w, g2_b):
    src3 = edge_index[0].reshape(NW, NCH, CH)
    dst3 = edge_index[1].reshape(NW, NCH, CH)
    mb = mlp_b.reshape(1, D)
    l1b = lin1_b.reshape(1, D)
    g1b = g1_b.reshape(1, D)
    l2b = lin2_b.reshape(1, D)
    g2b = g2_b.reshape(1, D)

    x, xw1, xh1 = _stage1(features, id_embedding, mlp_w.T, mb, conv1_w,
                          lin1_w.T, l1b)
    p1 = _sc_conv(xw1, src3, dst3)
    xw2, xh2 = _stage2(p1, xh1, id_embedding, g1_w.T, g1b, conv2_w,
                       lin2_w.T, l2b)
    p2 = _sc_conv(xw2, src3, dst3)
    return _stage3(p2, xh2, g2_w.T, g2b)


def kernel(features, id_embedding, edge_index, mlp_w, mlp_b, conv1_w,
           lin1_w, lin1_b, g1_w, g1_b, conv2_w, lin2_w, lin2_b, g2_w, g2_b):
    return _pipeline(features, id_embedding, edge_index, mlp_w, mlp_b, conv1_w,
                     lin1_w, lin1_b, g1_w, g1_b, conv2_w, lin2_w, lin2_b,
                     g2_w, g2_b)
